# baseline (speedup 1.0000x reference)
import functools

import jax
import jax.numpy as jnp
from jax import lax
from jax.experimental import pallas as pl
from jax.experimental.pallas import tpu as pltpu
from jax.experimental.pallas import tpu_sc as plsc

B = 8
N_CLS = 64
SEQ = 77
N_CTX = 4
D = 512
SUF = SEQ - 1 - N_CTX

_info = plsc.get_sparse_core_info()
_NC = _info.num_cores
_NS = _info.num_subcores
_NW = _NC * _NS
_CPW = N_CLS // _NW


def _mlp_body(im_ref, w1_ref, b1_ref, w2_ref, b2_ref, ctx_ref, ub_ref, out_ref):
    h = jnp.maximum(
        jnp.dot(im_ref[...], w1_ref[...], preferred_element_type=jnp.float32)
        + b1_ref[...],
        0.0,
    )
    bias = (
        jnp.dot(h, w2_ref[...], preferred_element_type=jnp.float32) + b2_ref[...]
    ) * ub_ref[0, 0]
    out_ref[...] = ctx_ref[...][None, :, :] + bias[:, None, :]


def _meta_net_ctx(im_features, W1, b1, W2, b2, ctx, use_bias):
    ub = jnp.asarray(use_bias, jnp.float32).reshape(1, 1)
    return pl.pallas_call(
        _mlp_body,
        out_shape=jax.ShapeDtypeStruct((B, N_CTX, D), jnp.float32),
    )(im_features, W1, b1.reshape(1, -1), W2, b2.reshape(1, -1), ctx, ub)


@functools.partial(
    pl.kernel,
    mesh=plsc.VectorSubcoreMesh(core_axis_name="c", subcore_axis_name="s"),
    compiler_params=pltpu.CompilerParams(use_tc_tiling_on_sc=False),
    out_type=jax.ShapeDtypeStruct((B, N_CLS, SEQ, D), jnp.float32),
    scratch_types=[
        pltpu.VMEM((_CPW, SEQ), jnp.int32),
        pltpu.VMEM((SEQ, D), jnp.float32),
        pltpu.VMEM((SEQ, D), jnp.float32),
        pltpu.VMEM((B, N_CTX, D), jnp.float32),
        pltpu.SemaphoreType.DMA,
        pltpu.SemaphoreType.DMA,
        pltpu.SemaphoreType.DMA,
    ],
)
def _sc_assemble(table_hbm, tok_hbm, ctxs_hbm, out_hbm,
                 idx_v, rows0, rows1, ctx_v, sem0, sem1, csem):
    wid = lax.axis_index("s") * _NC + lax.axis_index("c")
    c0 = wid * _CPW
    pltpu.sync_copy(tok_hbm.at[pl.ds(c0, _CPW)], idx_v)
    g0 = pltpu.async_copy(table_hbm.at[idx_v.at[0]], rows0, sem0)
    g1 = pltpu.async_copy(table_hbm.at[idx_v.at[1]], rows1, sem1)
    gc = pltpu.async_copy(ctxs_hbm, ctx_v, csem)
    gc.wait()

    def emit(rows, c):
        for b in range(B):
            pltpu.sync_copy(rows.at[pl.ds(0, 1)], out_hbm.at[b, c, pl.ds(0, 1)])
            pltpu.sync_copy(ctx_v.at[b], out_hbm.at[b, c, pl.ds(1, N_CTX)])
            pltpu.sync_copy(rows.at[pl.ds(1 + N_CTX, SUF)],
                            out_hbm.at[b, c, pl.ds(1 + N_CTX, SUF)])

    g0.wait()
    emit(rows0, c0)
    g1.wait()
    emit(rows1, c0 + 1)


def kernel(im_features, token_embedding, ctx, W1, b1, W2, b2,
           tokenized_prompts, use_bias=True):
    ctx_shifted = _meta_net_ctx(im_features, W1, b1, W2, b2, ctx, use_bias)
    special_prompts = _sc_assemble(token_embedding, tokenized_prompts,
                                   ctx_shifted)
    return (special_prompts, tokenized_prompts)


# speedup vs baseline: 1.8028x; 1.8028x over previous
import functools

import jax
import jax.numpy as jnp
from jax import lax
from jax.experimental import pallas as pl
from jax.experimental.pallas import tpu as pltpu
from jax.experimental.pallas import tpu_sc as plsc

B = 8
N_CLS = 64
SEQ = 77
N_CTX = 4
D = 512
LANES = 16
MAIN = 64
TAIL = 16
NTAIL = SEQ - MAIN

_info = plsc.get_sparse_core_info()
_NC = _info.num_cores
_NS = _info.num_subcores
_NW = _NC * _NS
_CPW = N_CLS // _NW


def _mlp_body(im_ref, w1_ref, b1_ref, w2_ref, b2_ref, ctx_ref, ub_ref, out_ref):
    h = jnp.maximum(
        jnp.dot(im_ref[...], w1_ref[...], preferred_element_type=jnp.float32)
        + b1_ref[...],
        0.0,
    )
    bias = (
        jnp.dot(h, w2_ref[...], preferred_element_type=jnp.float32) + b2_ref[...]
    ) * ub_ref[0, 0]
    out_ref[...] = ctx_ref[...][None, :, :] + bias[:, None, :]


def _meta_net_ctx(im_features, W1, b1, W2, b2, ctx, use_bias):
    ub = jnp.asarray(use_bias, jnp.float32).reshape(1, 1)
    return pl.pallas_call(
        _mlp_body,
        out_shape=jax.ShapeDtypeStruct((B, N_CTX, D), jnp.float32),
    )(im_features, W1, b1.reshape(1, -1), W2, b2.reshape(1, -1), ctx, ub)


@functools.partial(
    pl.kernel,
    mesh=plsc.VectorSubcoreMesh(core_axis_name="c", subcore_axis_name="s"),
    out_type=jax.ShapeDtypeStruct((B, N_CLS, SEQ, D), jnp.float32),
    scratch_types=[
        pltpu.VMEM((1, MAIN), jnp.int32),
        pltpu.VMEM((1, MAIN), jnp.int32),
        pltpu.VMEM((1, TAIL), jnp.int32),
        pltpu.VMEM((1, TAIL), jnp.int32),
        pltpu.VMEM((SEQ, D), jnp.float32),
        pltpu.VMEM((SEQ, D), jnp.float32),
        pltpu.VMEM((TAIL, D), jnp.float32),
        pltpu.VMEM((B, N_CTX, D), jnp.float32),
        pltpu.SemaphoreType.DMA,
        pltpu.SemaphoreType.DMA,
        pltpu.SemaphoreType.DMA,
        pltpu.SemaphoreType.DMA,
        pltpu.SemaphoreType.DMA,
    ],
)
def _sc_assemble(table_hbm, tok_main_hbm, tok_tail_hbm, ctxs_hbm, out_hbm,
                 idxm0, idxm1, idxt0, idxt1, buf_a, buf_b, tail_v, ctx_v,
                 gsem_a, gsem_b, tsem, ssem_a, ssem_b):
    wid = lax.axis_index("s") * _NC + lax.axis_index("c")
    c0 = wid * _CPW
    pltpu.sync_copy(tok_main_hbm.at[c0], idxm0)
    pltpu.sync_copy(tok_main_hbm.at[c0 + 1], idxm1)
    pltpu.sync_copy(tok_tail_hbm.at[c0], idxt0)
    pltpu.sync_copy(tok_tail_hbm.at[c0 + 1], idxt1)
    pltpu.sync_copy(ctxs_hbm, ctx_v)

    bufs = (buf_a, buf_b)
    gsems = (gsem_a, gsem_b)
    ssems = (ssem_a, ssem_b)

    def copy_rows(dst, dst_row0, src, src_row0, nrows):
        def body(k):
            for r in range(nrows):
                dst[dst_row0 + r, pl.ds(k * LANES, LANES)] = (
                    src[src_row0 + r, pl.ds(k * LANES, LANES)]
                )
        pl.loop(0, D // LANES)(body)

    for ci in range(_CPW):
        c = c0 + ci
        idxm = (idxm0, idxm1)[ci]
        idxt = (idxt0, idxt1)[ci]
        g0 = pltpu.async_copy(table_hbm.at[idxm.at[0]],
                              buf_a.at[pl.ds(0, MAIN)], gsem_a)
        g1 = pltpu.async_copy(table_hbm.at[idxm.at[0]],
                              buf_b.at[pl.ds(0, MAIN)], gsem_b)
        gt = pltpu.async_copy(table_hbm.at[idxt.at[0]], tail_v, tsem)
        gt.wait()
        gathers = [g0, g1]
        pending = [None, None]
        for b in range(B):
            p = b % 2
            if gathers[p] is not None:
                gathers[p].wait()
                gathers[p] = None
                copy_rows(bufs[p], MAIN, tail_v, 0, NTAIL)
            if pending[p] is not None:
                pending[p].wait()
            def ctx_body(k, buf=bufs[p], b=b):
                for j in range(N_CTX):
                    buf[1 + j, pl.ds(k * LANES, LANES)] = (
                        ctx_v[b, j, pl.ds(k * LANES, LANES)]
                    )
            pl.loop(0, D // LANES)(ctx_body)
            pending[p] = pltpu.async_copy(bufs[p], out_hbm.at[b, c], ssems[p])
        pending[0].wait()
        pending[1].wait()


def kernel(im_features, token_embedding, ctx, W1, b1, W2, b2,
           tokenized_prompts, use_bias=True):
    ctx_shifted = _meta_net_ctx(im_features, W1, b1, W2, b2, ctx, use_bias)
    tok_main = tokenized_prompts[:, :MAIN].reshape(N_CLS, 1, MAIN)
    tok_tail = jnp.pad(tokenized_prompts[:, MAIN:],
                       ((0, 0), (0, TAIL - NTAIL))).reshape(N_CLS, 1, TAIL)
    special_prompts = _sc_assemble(token_embedding, tok_main, tok_tail,
                                   ctx_shifted)
    return (special_prompts, tokenized_prompts)


# speedup vs baseline: 3.9967x; 2.2169x over previous
import functools

import jax
import jax.numpy as jnp
from jax import lax
from jax.experimental import pallas as pl
from jax.experimental.pallas import tpu as pltpu
from jax.experimental.pallas import tpu_sc as plsc

B = 8
N_CLS = 64
SEQ = 77
N_CTX = 4
D = 512
LANES = 16
NPLANES = SEQ - N_CTX

_info = plsc.get_sparse_core_info()
_NC = _info.num_cores
_NS = _info.num_subcores
_NW = _NC * _NS
_MAXP = -(-NPLANES // _NW)


def _mlp_body(im_ref, w1_ref, b1_ref, w2_ref, b2_ref, ctx_ref, ub_ref, out_ref):
    h = jnp.maximum(
        jnp.dot(im_ref[...], w1_ref[...], preferred_element_type=jnp.float32)
        + b1_ref[...],
        0.0,
    )
    bias = (
        jnp.dot(h, w2_ref[...], preferred_element_type=jnp.float32) + b2_ref[...]
    ) * ub_ref[0, 0]
    out_ref[...] = ctx_ref[...][None, :, :] + bias[:, None, :]


def _meta_net_ctx(im_features, W1, b1, W2, b2, ctx, use_bias):
    ub = jnp.asarray(use_bias, jnp.float32).reshape(1, 1)
    return pl.pallas_call(
        _mlp_body,
        out_shape=jax.ShapeDtypeStruct((B, N_CTX, D), jnp.float32),
    )(im_features, W1, b1.reshape(1, -1), W2, b2.reshape(1, -1), ctx, ub)


@functools.partial(
    pl.kernel,
    mesh=plsc.VectorSubcoreMesh(core_axis_name="c", subcore_axis_name="s"),
    out_type=jax.ShapeDtypeStruct((B, SEQ, N_CLS, D), jnp.float32),
    scratch_types=[
        pltpu.VMEM((1, N_CLS), jnp.int32),
        pltpu.VMEM((1, D), jnp.float32),
        pltpu.VMEM((N_CLS, D), jnp.float32),
        pltpu.VMEM((N_CLS, D), jnp.float32),
        pltpu.SemaphoreType.DMA,
        pltpu.SemaphoreType.DMA,
        pltpu.SemaphoreType.DMA,
    ],
)
def _sc_assemble(table_hbm, tok_hbm, ctxs_hbm, out_hbm,
                 idx_v, ctxrow_v, buf_a, buf_b,
                 gsem, ssem_a, ssem_b):
    wid = lax.axis_index("s") * _NC + lax.axis_index("c")

    pltpu.sync_copy(ctxs_hbm.at[wid], ctxrow_v)

    def fill_row(r):
        def chunk(k):
            buf_a[r, pl.ds(k * LANES, LANES)] = ctxrow_v[0, pl.ds(k * LANES, LANES)]
        pl.loop(0, D // LANES)(chunk)

    pl.loop(0, N_CLS)(fill_row)
    b_ctx = wid // N_CTX
    j_ctx = wid % N_CTX
    ctx_store = pltpu.async_copy(buf_a, out_hbm.at[b_ctx, 1 + j_ctx], ssem_a)

    bufs = (buf_a, buf_b)
    ssems = (ssem_a, ssem_b)
    outstanding = [1, 0]
    for i in range(_MAXP):
        p = wid + _NW * i
        slot = (i + 1) % 2
        buf = bufs[slot]
        sem = ssems[slot]

        @pl.when(p < NPLANES)
        def _():
            s = jnp.where(p == 0, 0, p + N_CTX)
            pltpu.sync_copy(tok_hbm.at[s], idx_v)
            for _ in range(outstanding[slot]):
                pltpu.make_async_copy(buf, out_hbm.at[0, 0], sem).wait()
            pltpu.async_copy(table_hbm.at[idx_v.at[0]], buf, gsem).wait()
            for b in range(B):
                pltpu.async_copy(buf, out_hbm.at[b, s], sem)

        outstanding[slot] = B

    for slot in (0, 1):
        for _ in range(outstanding[slot]):
            pltpu.make_async_copy(bufs[slot], out_hbm.at[0, 0], ssems[slot]).wait()


def kernel(im_features, token_embedding, ctx, W1, b1, W2, b2,
           tokenized_prompts, use_bias=True):
    ctx_shifted = _meta_net_ctx(im_features, W1, b1, W2, b2, ctx, use_bias)
    ctxs2 = ctx_shifted.reshape(B * N_CTX, 1, D)
    tok_t = tokenized_prompts.T.reshape(SEQ, 1, N_CLS)
    out_t = _sc_assemble(token_embedding, tok_t, ctxs2)
    special_prompts = jnp.transpose(out_t, (0, 2, 1, 3))
    return (special_prompts, tokenized_prompts)


# speedup vs baseline: 4.0591x; 1.0156x over previous
import functools

import jax
import jax.numpy as jnp
from jax import lax
from jax.experimental import pallas as pl
from jax.experimental.pallas import tpu as pltpu
from jax.experimental.pallas import tpu_sc as plsc

B = 8
N_CLS = 64
SEQ = 77
N_CTX = 4
D = 512
LANES = 16
NPLANES = SEQ - N_CTX

_info = plsc.get_sparse_core_info()
_NC = _info.num_cores
_NS = _info.num_subcores
_NW = _NC * _NS
_MAXP = -(-NPLANES // _NW)


def _mlp_body(im_ref, w1_ref, b1_ref, w2_ref, b2_ref, ctx_ref, ub_ref, out_ref):
    h = jnp.maximum(
        jnp.dot(im_ref[...], w1_ref[...], preferred_element_type=jnp.float32)
        + b1_ref[...],
        0.0,
    )
    bias = (
        jnp.dot(h, w2_ref[...], preferred_element_type=jnp.float32) + b2_ref[...]
    ) * ub_ref[0, 0]
    out_ref[...] = ctx_ref[...][None, :, :] + bias[:, None, :]


def _meta_net_ctx(im_features, W1, b1, W2, b2, ctx, use_bias):
    ub = jnp.asarray(use_bias, jnp.float32).reshape(1, 1)
    return pl.pallas_call(
        _mlp_body,
        out_shape=jax.ShapeDtypeStruct((B, N_CTX, D), jnp.float32),
    )(im_features, W1, b1.reshape(1, -1), W2, b2.reshape(1, -1), ctx, ub)


@functools.partial(
    pl.kernel,
    mesh=plsc.VectorSubcoreMesh(core_axis_name="c", subcore_axis_name="s"),
    out_type=jax.ShapeDtypeStruct((B, SEQ, N_CLS, D), jnp.float32),
    scratch_types=[
        pltpu.VMEM((1, N_CLS), jnp.int32),
        pltpu.VMEM((1, D), jnp.float32),
        pltpu.VMEM((N_CLS, D), jnp.float32),
        pltpu.VMEM((N_CLS, D), jnp.float32),
        pltpu.SemaphoreType.DMA,
        pltpu.SemaphoreType.DMA,
        pltpu.SemaphoreType.DMA,
    ],
)
def _sc_assemble(table_hbm, tok_hbm, ctxs_hbm, out_hbm,
                 idx_v, ctxrow_v, buf_a, buf_b,
                 gsem, ssem_a, ssem_b):
    wid = lax.axis_index("s") * _NC + lax.axis_index("c")

    pltpu.sync_copy(ctxs_hbm.at[wid], ctxrow_v)

    def fill_row(r):
        def chunk(k):
            buf_a[r, pl.ds(k * LANES, LANES)] = ctxrow_v[0, pl.ds(k * LANES, LANES)]
        pl.loop(0, D // LANES)(chunk)

    pl.loop(0, N_CLS)(fill_row)
    b_ctx = wid // N_CTX
    j_ctx = wid % N_CTX
    ctx_store = pltpu.async_copy(buf_a, out_hbm.at[b_ctx, 1 + j_ctx], ssem_a)

    bufs = (buf_a, buf_b)
    ssems = (ssem_a, ssem_b)
    tot = NPLANES * B
    lo = (tot * wid) // _NW
    hi = (tot * (wid + 1)) // _NW
    p_base = lo // B
    outstanding = [jnp.int32(1), jnp.int32(0)]
    for i in range(4):
        p = p_base + i
        t_lo = jnp.maximum(lo, p * B)
        t_hi = jnp.minimum(hi, (p + 1) * B)
        n_i = jnp.maximum(t_hi - t_lo, 0)
        slot = i % 2
        buf = bufs[slot]
        sem = ssems[slot]

        def drain(_, buf=buf, sem=sem):
            pltpu.make_async_copy(buf, out_hbm.at[0, 0], sem).wait()
        pl.loop(0, outstanding[slot])(drain)
        outstanding[slot] = n_i

        @pl.when(n_i > 0)
        def _(p=p, t_lo=t_lo, t_hi=t_hi, buf=buf, sem=sem):
            s = jnp.where(p == 0, 0, p + N_CTX)
            pltpu.sync_copy(tok_hbm.at[s], idx_v)
            pltpu.async_copy(table_hbm.at[idx_v.at[0]], buf, gsem).wait()
            def store(b, buf=buf, sem=sem, s=s):
                pltpu.async_copy(buf, out_hbm.at[b, s], sem)
            pl.loop(t_lo - p * B, t_hi - p * B)(store)

    for slot in (0, 1):
        def drain(_, slot=slot):
            pltpu.make_async_copy(bufs[slot], out_hbm.at[0, 0], ssems[slot]).wait()
        pl.loop(0, outstanding[slot])(drain)


def kernel(im_features, token_embedding, ctx, W1, b1, W2, b2,
           tokenized_prompts, use_bias=True):
    ctx_shifted = _meta_net_ctx(im_features, W1, b1, W2, b2, ctx, use_bias)
    ctxs2 = ctx_shifted.reshape(B * N_CTX, 1, D)
    tok_t = tokenized_prompts.T.reshape(SEQ, 1, N_CLS)
    out_t = _sc_assemble(token_embedding, tok_t, ctxs2)
    special_prompts = jnp.transpose(out_t, (0, 2, 1, 3))
    return (special_prompts, tokenized_prompts)


# speedup vs baseline: 4.1690x; 1.0271x over previous
import functools

import jax
import jax.numpy as jnp
from jax import lax
from jax.experimental import pallas as pl
from jax.experimental.pallas import tpu as pltpu
from jax.experimental.pallas import tpu_sc as plsc

B = 8
N_CLS = 64
SEQ = 77
N_CTX = 4
D = 512
LANES = 16
NPLANES = SEQ - N_CTX

_info = plsc.get_sparse_core_info()
_NC = _info.num_cores
_NS = _info.num_subcores
_NW = _NC * _NS
_MAXP = -(-NPLANES // _NW)


def _mlp_body(im_ref, w1_ref, b1_ref, w2_ref, b2_ref, ctx_ref, ub_ref, out_ref):
    h = jnp.maximum(
        jnp.dot(im_ref[...], w1_ref[...], preferred_element_type=jnp.float32)
        + b1_ref[...],
        0.0,
    )
    bias = (
        jnp.dot(h, w2_ref[...], preferred_element_type=jnp.float32) + b2_ref[...]
    ) * ub_ref[0, 0]
    out = ctx_ref[...][None, :, :] + bias[:, None, :]
    out_ref[...] = out.reshape(B * N_CTX, 1, D)


def _meta_net_ctx(im_features, W1, b1, W2, b2, ctx, use_bias):
    ub = jnp.asarray(use_bias, jnp.float32).reshape(1, 1)
    return pl.pallas_call(
        _mlp_body,
        out_shape=jax.ShapeDtypeStruct((B * N_CTX, 1, D), jnp.float32),
    )(im_features, W1, b1.reshape(1, -1), W2, b2.reshape(1, -1), ctx, ub)


@functools.partial(
    pl.kernel,
    mesh=plsc.VectorSubcoreMesh(core_axis_name="c", subcore_axis_name="s"),
    out_type=jax.ShapeDtypeStruct((B, SEQ, N_CLS, D), jnp.float32),
    scratch_types=[
        pltpu.VMEM((1, N_CLS), jnp.int32),
        pltpu.VMEM((1, D), jnp.float32),
        pltpu.VMEM((N_CLS, D), jnp.float32),
        pltpu.VMEM((N_CLS, D), jnp.float32),
        pltpu.SemaphoreType.DMA,
        pltpu.SemaphoreType.DMA,
        pltpu.SemaphoreType.DMA,
    ],
)
def _sc_assemble(table_hbm, tok_hbm, ctxs_hbm, out_hbm,
                 idx_v, ctxrow_v, buf_a, buf_b,
                 gsem, ssem_a, ssem_b):
    wid = lax.axis_index("s") * _NC + lax.axis_index("c")

    pltpu.sync_copy(ctxs_hbm.at[wid], ctxrow_v)

    def fill_row(r):
        def chunk(k):
            buf_a[r, pl.ds(k * LANES, LANES)] = ctxrow_v[0, pl.ds(k * LANES, LANES)]
        pl.loop(0, D // LANES)(chunk)

    pl.loop(0, N_CLS)(fill_row)
    b_ctx = wid // N_CTX
    j_ctx = wid % N_CTX
    ctx_store = pltpu.async_copy(buf_a, out_hbm.at[b_ctx, 1 + j_ctx], ssem_a)

    bufs = (buf_a, buf_b)
    ssems = (ssem_a, ssem_b)
    tot = NPLANES * B
    lo = (tot * wid) // _NW
    hi = (tot * (wid + 1)) // _NW
    p_base = lo // B
    outstanding = [jnp.int32(1), jnp.int32(0)]
    for i in range(4):
        p = p_base + i
        t_lo = jnp.maximum(lo, p * B)
        t_hi = jnp.minimum(hi, (p + 1) * B)
        n_i = jnp.maximum(t_hi - t_lo, 0)
        slot = i % 2
        buf = bufs[slot]
        sem = ssems[slot]

        def drain(_, buf=buf, sem=sem):
            pltpu.make_async_copy(buf, out_hbm.at[0, 0], sem).wait()
        pl.loop(0, outstanding[slot])(drain)
        outstanding[slot] = n_i

        @pl.when(n_i > 0)
        def _(p=p, t_lo=t_lo, t_hi=t_hi, buf=buf, sem=sem):
            s = jnp.where(p == 0, 0, p + N_CTX)
            pltpu.sync_copy(tok_hbm.at[s], idx_v)
            pltpu.async_copy(table_hbm.at[idx_v.at[0]], buf, gsem).wait()
            def store(b, buf=buf, sem=sem, s=s):
                pltpu.async_copy(buf, out_hbm.at[b, s], sem)
            pl.loop(t_lo - p * B, t_hi - p * B)(store)

    for slot in (0, 1):
        def drain(_, slot=slot):
            pltpu.make_async_copy(bufs[slot], out_hbm.at[0, 0], ssems[slot]).wait()
        pl.loop(0, outstanding[slot])(drain)


def kernel(im_features, token_embedding, ctx, W1, b1, W2, b2,
           tokenized_prompts, use_bias=True):
    ctxs2 = _meta_net_ctx(im_features, W1, b1, W2, b2, ctx, use_bias)
    tok_t = tokenized_prompts.T.reshape(SEQ, 1, N_CLS)
    out_t = _sc_assemble(token_embedding, tok_t, ctxs2)
    special_prompts = jnp.transpose(out_t, (0, 2, 1, 3))
    return (special_prompts, tokenized_prompts)
